# initial kernel scaffold (unmeasured)
import jax
import jax.numpy as jnp
from jax import lax
from jax.experimental import pallas as pl
from jax.experimental.pallas import tpu as pltpu

N_DEV = 4


def kernel(x, w_mat):
    m, k_per = x.shape
    _, n = w_mat.shape

    def body(x_ref, w_ref, out_ref, comm_ref, send_sems, recv_sems):
        my = lax.axis_index("i")
        left = lax.rem(my + N_DEV - 1, N_DEV)
        right = lax.rem(my + 1, N_DEV)

        barrier_sem = pltpu.get_barrier_semaphore()
        for nbr in (left, right):
            pl.semaphore_signal(
                barrier_sem, inc=1,
                device_id=(nbr,), device_id_type=pl.DeviceIdType.MESH,
            )
        pl.semaphore_wait(barrier_sem, 2)

        part = jnp.dot(
            x_ref[:, :].astype(jnp.bfloat16),
            w_ref[:, :].astype(jnp.bfloat16),
            preferred_element_type=jnp.float32,
        )
        out_ref[:, :] = part
        comm_ref[0, :, :] = part.astype(jnp.bfloat16)

        for h in range(N_DEV - 1):
            rdma = pltpu.make_async_remote_copy(
                src_ref=comm_ref.at[h],
                dst_ref=comm_ref.at[h + 1],
                send_sem=send_sems.at[h],
                recv_sem=recv_sems.at[h],
                device_id=(right,),
                device_id_type=pl.DeviceIdType.MESH,
            )
            rdma.start()
            rdma.wait()
            out_ref[:, :] += comm_ref[h + 1, :, :].astype(jnp.float32)

        y = out_ref[:, :]
        out_ref[:, :] = y * (1.0 / (1.0 + jnp.exp(-y)))

    return pl.pallas_call(
        body,
        out_shape=jax.ShapeDtypeStruct((m, n), jnp.float32),
        in_specs=[
            pl.BlockSpec(memory_space=pltpu.VMEM),
            pl.BlockSpec(memory_space=pltpu.VMEM),
        ],
        out_specs=pl.BlockSpec(memory_space=pltpu.VMEM),
        scratch_shapes=[
            pltpu.VMEM((N_DEV, m, n), jnp.bfloat16),
            pltpu.SemaphoreType.DMA((N_DEV - 1,)),
            pltpu.SemaphoreType.DMA((N_DEV - 1,)),
        ],
        compiler_params=pltpu.CompilerParams(collective_id=0),
    )(x, w_mat)


# baseline (device time: 313422 ns/iter reference)
import jax
import jax.numpy as jnp
from jax import lax
from jax.experimental import pallas as pl
from jax.experimental.pallas import tpu as pltpu

N_DEV = 4


def kernel(x, w_mat):
    m, k_per = x.shape
    _, n = w_mat.shape

    def body(x_ref, w_ref, out_ref, comm_ref, send_sems, recv_sems):
        my = lax.axis_index("i")
        left = lax.rem(my + N_DEV - 1, N_DEV)
        right = lax.rem(my + 1, N_DEV)

        barrier_sem = pltpu.get_barrier_semaphore()
        for nbr in (left, right):
            pl.semaphore_signal(
                barrier_sem, inc=1,
                device_id=(nbr,), device_id_type=pl.DeviceIdType.MESH,
            )
        pl.semaphore_wait(barrier_sem, 2)

        part = jnp.dot(
            x_ref[:, :].astype(jnp.bfloat16),
            w_ref[:, :].astype(jnp.bfloat16),
            preferred_element_type=jnp.float32,
        )
        out_ref[:, :] = part
        comm_ref[0, :, :] = part.astype(jnp.bfloat16)

        for h in range(N_DEV - 1):
            rdma = pltpu.make_async_remote_copy(
                src_ref=comm_ref.at[h],
                dst_ref=comm_ref.at[h + 1],
                send_sem=send_sems.at[h],
                recv_sem=recv_sems.at[h],
                device_id=(right,),
                device_id_type=pl.DeviceIdType.MESH,
            )
            rdma.start()
            rdma.wait()
            out_ref[:, :] += comm_ref[h + 1, :, :].astype(jnp.float32)

        y = out_ref[:, :]
        out_ref[:, :] = y * (1.0 / (1.0 + jnp.exp(-y)))

    return pl.pallas_call(
        body,
        out_shape=jax.ShapeDtypeStruct((m, n), jnp.float32),
        in_specs=[
            pl.BlockSpec(memory_space=pltpu.VMEM),
            pl.BlockSpec(memory_space=pltpu.VMEM),
        ],
        out_specs=pl.BlockSpec(memory_space=pltpu.VMEM),
        scratch_shapes=[
            pltpu.VMEM((N_DEV, m, n), jnp.bfloat16),
            pltpu.SemaphoreType.DMA((N_DEV - 1,)),
            pltpu.SemaphoreType.DMA((N_DEV - 1,)),
        ],
        compiler_params=pltpu.CompilerParams(
            collective_id=0,
            vmem_limit_bytes=128 * 1024 * 1024,
        ),
    )(x, w_mat)


# device time: 109657 ns/iter; 2.8582x vs baseline; 2.8582x over previous
import jax
import jax.numpy as jnp
from jax import lax
from jax.experimental import pallas as pl
from jax.experimental.pallas import tpu as pltpu

N_DEV = 4
F32 = jnp.float32
BF16 = jnp.bfloat16


def kernel(x, w_mat):
    m, k_per = x.shape
    _, n = w_mat.shape
    h2 = m // 4
    h4 = m // 8

    def _silu(y):
        return y * (1.0 / (1.0 + jnp.exp(-y)))

    def body(x_ref, w_ref, out_ref, work, ru1, ru2, rv1, rv2, ss, rs):
        my = lax.axis_index("i")
        yp = my ^ 1
        xp = my ^ 3
        a = my // 2
        b = (my ^ (my // 2)) & 1

        barrier_sem = pltpu.get_barrier_semaphore()
        for nbr in (yp, xp):
            pl.semaphore_signal(
                barrier_sem, inc=1,
                device_id=(nbr,), device_id_type=pl.DeviceIdType.MESH,
            )
        pl.semaphore_wait(barrier_sem, 2)

        part = jnp.dot(
            x_ref[:, :].astype(BF16),
            w_ref[:, :].astype(BF16),
            preferred_element_type=F32,
        )
        work[:, :] = part.astype(BF16)

        u_keep = b * h2
        u_send = (1 - b) * h2
        uq_keep = u_keep + a * h4
        uq_send = u_keep + (1 - a) * h4
        v_base = m // 2
        v_keep = v_base + a * h2
        v_send = v_base + (1 - a) * h2
        vq_keep = v_keep + b * h4
        vq_send = v_keep + (1 - b) * h4

        def xchg(src_rows, nrows, dst_ref, sem_idx, partner):
            return pltpu.make_async_remote_copy(
                src_ref=work.at[pl.ds(src_rows, nrows)],
                dst_ref=dst_ref,
                send_sem=ss.at[sem_idx],
                recv_sem=rs.at[sem_idx],
                device_id=(partner,),
                device_id_type=pl.DeviceIdType.MESH,
            )

        def reduce_rows(off, nrows, recv):
            cur = work[pl.ds(off, nrows), :].astype(F32)
            work[pl.ds(off, nrows), :] = (
                cur + recv[:, :].astype(F32)
            ).astype(BF16)

        su = xchg(u_send, h2, ru1, 0, yp)
        sv = xchg(v_send, h2, rv1, 4, xp)
        su.start()
        sv.start()
        su.wait()
        reduce_rows(u_keep, h2, ru1)
        sv.wait()
        reduce_rows(v_keep, h2, rv1)

        su = xchg(uq_send, h4, ru2, 1, xp)
        sv = xchg(vq_send, h4, rv2, 5, yp)
        su.start()
        sv.start()
        su.wait()
        reduce_rows(uq_keep, h4, ru2)
        sv.wait()
        reduce_rows(vq_keep, h4, rv2)

        su = xchg(uq_keep, h4, work.at[pl.ds(uq_keep, h4)], 2, xp)
        sv = xchg(vq_keep, h4, work.at[pl.ds(vq_keep, h4)], 6, yp)
        su.start()
        sv.start()
        ru = xchg(uq_keep, h4, work.at[pl.ds(uq_send, h4)], 2, xp)
        rv = xchg(vq_keep, h4, work.at[pl.ds(vq_send, h4)], 6, yp)
        ru.wait_recv()
        rv.wait_recv()
        su.wait_send()
        sv.wait_send()

        su = xchg(u_keep, h2, work.at[pl.ds(u_keep, h2)], 3, yp)
        sv = xchg(v_keep, h2, work.at[pl.ds(v_keep, h2)], 7, xp)
        su.start()
        sv.start()
        ru = xchg(u_keep, h2, work.at[pl.ds(u_send, h2)], 3, yp)
        rv = xchg(v_keep, h2, work.at[pl.ds(v_send, h2)], 7, xp)
        ru.wait_recv()
        yu = work[0 : 2 * h2, :].astype(F32)
        out_ref[0 : 2 * h2, :] = _silu(yu)
        rv.wait_recv()
        yv = work[2 * h2 : 4 * h2, :].astype(F32)
        out_ref[2 * h2 : 4 * h2, :] = _silu(yv)
        su.wait_send()
        sv.wait_send()

    return pl.pallas_call(
        body,
        out_shape=jax.ShapeDtypeStruct((m, n), F32),
        in_specs=[
            pl.BlockSpec(memory_space=pltpu.VMEM),
            pl.BlockSpec(memory_space=pltpu.VMEM),
        ],
        out_specs=pl.BlockSpec(memory_space=pltpu.VMEM),
        scratch_shapes=[
            pltpu.VMEM((m, n), BF16),
            pltpu.VMEM((h2, n), BF16),
            pltpu.VMEM((h4, n), BF16),
            pltpu.VMEM((h2, n), BF16),
            pltpu.VMEM((h4, n), BF16),
            pltpu.SemaphoreType.DMA((8,)),
            pltpu.SemaphoreType.DMA((8,)),
        ],
        compiler_params=pltpu.CompilerParams(
            collective_id=0,
            vmem_limit_bytes=128 * 1024 * 1024,
        ),
    )(x, w_mat)


# device time: 102123 ns/iter; 3.0691x vs baseline; 1.0738x over previous
import jax
import jax.numpy as jnp
from jax import lax
from jax.experimental import pallas as pl
from jax.experimental.pallas import tpu as pltpu

N_DEV = 4
F32 = jnp.float32
BF16 = jnp.bfloat16


def kernel(x, w_mat):
    m, k_per = x.shape
    _, n = w_mat.shape
    h2 = m // 4
    h4 = m // 8

    def body(x_ref, w_ref, out_ref, work, ru1, ru2, rv1, rv2, ss, rs):
        my = lax.axis_index("i")
        yp = my ^ 1
        xp = my ^ 3
        a = my // 2
        b = (my ^ (my // 2)) & 1

        barrier_sem = pltpu.get_barrier_semaphore()
        for nbr in (yp, xp):
            pl.semaphore_signal(
                barrier_sem, inc=1,
                device_id=(nbr,), device_id_type=pl.DeviceIdType.MESH,
            )
        pl.semaphore_wait(barrier_sem, 2)

        u_keep = b * h2
        u_send = (1 - b) * h2
        uq_keep = u_keep + a * h4
        uq_send = u_keep + (1 - a) * h4
        v_base = m // 2
        v_keep = v_base + a * h2
        v_send = v_base + (1 - a) * h2
        vq_keep = v_keep + b * h4
        vq_send = v_keep + (1 - b) * h4

        w_bf = w_ref[:, :].astype(BF16)

        def gemm_rows(off, nrows):
            work[pl.ds(off, nrows), :] = jnp.dot(
                x_ref[pl.ds(off, nrows), :].astype(BF16),
                w_bf,
                preferred_element_type=F32,
            ).astype(BF16)

        def xchg(src_rows, nrows, dst_ref, sem_idx, partner):
            return pltpu.make_async_remote_copy(
                src_ref=work.at[pl.ds(src_rows, nrows)],
                dst_ref=dst_ref,
                send_sem=ss.at[sem_idx],
                recv_sem=rs.at[sem_idx],
                device_id=(partner,),
                device_id_type=pl.DeviceIdType.MESH,
            )

        def reduce_rows(off, nrows, recv):
            cur = work[pl.ds(off, nrows), :].astype(F32)
            work[pl.ds(off, nrows), :] = (
                cur + recv[:, :].astype(F32)
            ).astype(BF16)

        def silu_rows(off, nrows):
            y = work[pl.ds(off, nrows), :].astype(F32)
            out_ref[pl.ds(off, nrows), :] = y * (1.0 / (1.0 + jnp.exp(-y)))

        gemm_rows(u_send, h2)
        su1 = xchg(u_send, h2, ru1, 0, yp)
        su1.start()
        gemm_rows(v_send, h2)
        sv1 = xchg(v_send, h2, rv1, 4, xp)
        sv1.start()
        gemm_rows(u_keep, h2)
        gemm_rows(v_keep, h2)

        su1.wait()
        reduce_rows(uq_send, h4, ru1.at[pl.ds((1 - a) * h4, h4)])
        su2 = xchg(uq_send, h4, ru2, 1, xp)
        su2.start()
        reduce_rows(uq_keep, h4, ru1.at[pl.ds(a * h4, h4)])
        sv1.wait()
        reduce_rows(vq_send, h4, rv1.at[pl.ds((1 - b) * h4, h4)])
        sv2 = xchg(vq_send, h4, rv2, 5, yp)
        sv2.start()
        reduce_rows(vq_keep, h4, rv1.at[pl.ds(b * h4, h4)])

        su2.wait()
        reduce_rows(uq_keep, h4, ru2)
        su3 = xchg(uq_keep, h4, work.at[pl.ds(uq_keep, h4)], 2, xp)
        su3.start()
        sv2.wait()
        reduce_rows(vq_keep, h4, rv2)
        sv3 = xchg(vq_keep, h4, work.at[pl.ds(vq_keep, h4)], 6, yp)
        sv3.start()
        silu_rows(uq_keep, h4)
        silu_rows(vq_keep, h4)

        ru3 = xchg(uq_keep, h4, work.at[pl.ds(uq_send, h4)], 2, xp)
        rv3 = xchg(vq_keep, h4, work.at[pl.ds(vq_send, h4)], 6, yp)
        ru3.wait_recv()
        su4 = xchg(u_keep, h2, work.at[pl.ds(u_keep, h2)], 3, yp)
        su4.start()
        silu_rows(uq_send, h4)
        rv3.wait_recv()
        sv4 = xchg(v_keep, h2, work.at[pl.ds(v_keep, h2)], 7, xp)
        sv4.start()
        silu_rows(vq_send, h4)

        ru4 = xchg(u_keep, h2, work.at[pl.ds(u_send, h2)], 3, yp)
        rv4 = xchg(v_keep, h2, work.at[pl.ds(v_send, h2)], 7, xp)
        ru4.wait_recv()
        silu_rows(u_send, h2)
        rv4.wait_recv()
        silu_rows(v_send, h2)

        su3.wait_send()
        sv3.wait_send()
        su4.wait_send()
        sv4.wait_send()

    return pl.pallas_call(
        body,
        out_shape=jax.ShapeDtypeStruct((m, n), F32),
        in_specs=[
            pl.BlockSpec(memory_space=pltpu.VMEM),
            pl.BlockSpec(memory_space=pltpu.VMEM),
        ],
        out_specs=pl.BlockSpec(memory_space=pltpu.VMEM),
        scratch_shapes=[
            pltpu.VMEM((m, n), BF16),
            pltpu.VMEM((h2, n), BF16),
            pltpu.VMEM((h4, n), BF16),
            pltpu.VMEM((h2, n), BF16),
            pltpu.VMEM((h4, n), BF16),
            pltpu.SemaphoreType.DMA((8,)),
            pltpu.SemaphoreType.DMA((8,)),
        ],
        compiler_params=pltpu.CompilerParams(
            collective_id=0,
            vmem_limit_bytes=128 * 1024 * 1024,
        ),
    )(x, w_mat)


# device time: 101287 ns/iter; 3.0944x vs baseline; 1.0083x over previous
import jax
import jax.numpy as jnp
from jax import lax
from jax.experimental import pallas as pl
from jax.experimental.pallas import tpu as pltpu

N_DEV = 4
F32 = jnp.float32
BF16 = jnp.bfloat16


def kernel(x, w_mat):
    m, k_per = x.shape
    _, n = w_mat.shape
    h2 = m // 4
    h4 = m // 8

    def body(x_ref, w_ref, out_ref, work, ru1, ru2, rv1, rv2, ss, rs):
        my = lax.axis_index("i")
        yp = my ^ 1
        xp = my ^ 3
        a = my // 2
        b = (my ^ (my // 2)) & 1

        barrier_sem = pltpu.get_barrier_semaphore()
        for nbr in (yp, xp):
            pl.semaphore_signal(
                barrier_sem, inc=1,
                device_id=(nbr,), device_id_type=pl.DeviceIdType.MESH,
            )
        pl.semaphore_wait(barrier_sem, 2)

        u_keep = b * h2
        u_send = (1 - b) * h2
        uq_keep = u_keep + a * h4
        uq_send = u_keep + (1 - a) * h4
        v_base = m // 2
        v_keep = v_base + a * h2
        v_send = v_base + (1 - a) * h2
        vq_keep = v_keep + b * h4
        vq_send = v_keep + (1 - b) * h4

        w_bf = w_ref[:, :].astype(BF16)

        def gemm_rows(off, nrows):
            work[pl.ds(off, nrows), :] = jnp.dot(
                x_ref[pl.ds(off, nrows), :].astype(BF16),
                w_bf,
                preferred_element_type=F32,
            ).astype(BF16)

        def copy(src_rows, nrows, dst_ref, sem_idx, partner):
            return pltpu.make_async_remote_copy(
                src_ref=work.at[pl.ds(src_rows, nrows)],
                dst_ref=dst_ref,
                send_sem=ss.at[sem_idx],
                recv_sem=rs.at[sem_idx],
                device_id=(partner,),
                device_id_type=pl.DeviceIdType.MESH,
            )

        def reduce_rows(off, nrows, recv):
            cur = work[pl.ds(off, nrows), :].astype(F32)
            work[pl.ds(off, nrows), :] = (
                cur + recv[:, :].astype(F32)
            ).astype(BF16)

        def silu_rows(off, nrows):
            y = work[pl.ds(off, nrows), :].astype(F32)
            out_ref[pl.ds(off, nrows), :] = y * (1.0 / (1.0 + jnp.exp(-y)))

        gemm_rows(u_send, h2)
        s1a = copy(u_send + (1 - a) * h4, h4, ru1.at[pl.ds((1 - a) * h4, h4)], 0, yp)
        s1b = copy(u_send + a * h4, h4, ru1.at[pl.ds(a * h4, h4)], 1, yp)
        s1a.start()
        s1b.start()
        gemm_rows(v_send, h2)
        t1a = copy(v_send + (1 - b) * h4, h4, rv1.at[pl.ds((1 - b) * h4, h4)], 6, xp)
        t1b = copy(v_send + b * h4, h4, rv1.at[pl.ds(b * h4, h4)], 7, xp)
        t1a.start()
        t1b.start()
        gemm_rows(u_keep, h2)
        gemm_rows(v_keep, h2)

        s1a.wait_recv()
        reduce_rows(uq_send, h4, ru1.at[pl.ds((1 - a) * h4, h4)])
        s2 = copy(uq_send, h4, ru2, 2, xp)
        s2.start()
        t1a.wait_recv()
        reduce_rows(vq_send, h4, rv1.at[pl.ds((1 - b) * h4, h4)])
        t2 = copy(vq_send, h4, rv2, 8, yp)
        t2.start()
        s1b.wait_recv()
        reduce_rows(uq_keep, h4, ru1.at[pl.ds(a * h4, h4)])
        t1b.wait_recv()
        reduce_rows(vq_keep, h4, rv1.at[pl.ds(b * h4, h4)])

        s2.wait_recv()
        reduce_rows(uq_keep, h4, ru2)
        s3 = copy(uq_keep, h4, work.at[pl.ds(uq_keep, h4)], 3, xp)
        s4a = copy(uq_keep, h4, work.at[pl.ds(uq_keep, h4)], 4, yp)
        s3.start()
        s4a.start()
        t2.wait_recv()
        reduce_rows(vq_keep, h4, rv2)
        t3 = copy(vq_keep, h4, work.at[pl.ds(vq_keep, h4)], 9, yp)
        t4a = copy(vq_keep, h4, work.at[pl.ds(vq_keep, h4)], 10, xp)
        t3.start()
        t4a.start()
        silu_rows(uq_keep, h4)
        silu_rows(vq_keep, h4)

        r3 = copy(uq_keep, h4, work.at[pl.ds(uq_send, h4)], 3, xp)
        r3.wait_recv()
        s4b = copy(uq_send, h4, work.at[pl.ds(uq_send, h4)], 5, yp)
        s4b.start()
        silu_rows(uq_send, h4)
        q3 = copy(vq_keep, h4, work.at[pl.ds(vq_send, h4)], 9, yp)
        q3.wait_recv()
        t4b = copy(vq_send, h4, work.at[pl.ds(vq_send, h4)], 11, xp)
        t4b.start()
        silu_rows(vq_send, h4)

        r4a = copy(uq_keep, h4, work.at[pl.ds(u_send + a * h4, h4)], 4, yp)
        q4a = copy(vq_keep, h4, work.at[pl.ds(v_send + b * h4, h4)], 10, xp)
        r4b = copy(uq_keep, h4, work.at[pl.ds(u_send + (1 - a) * h4, h4)], 5, yp)
        q4b = copy(vq_keep, h4, work.at[pl.ds(v_send + (1 - b) * h4, h4)], 11, xp)
        r4a.wait_recv()
        silu_rows(u_send + a * h4, h4)
        q4a.wait_recv()
        silu_rows(v_send + b * h4, h4)
        r4b.wait_recv()
        silu_rows(u_send + (1 - a) * h4, h4)
        q4b.wait_recv()
        silu_rows(v_send + (1 - b) * h4, h4)

        for d in (s1a, s1b, t1a, t1b, s2, t2, s3, s4a, t3, t4a, s4b, t4b):
            d.wait_send()

    return pl.pallas_call(
        body,
        out_shape=jax.ShapeDtypeStruct((m, n), F32),
        in_specs=[
            pl.BlockSpec(memory_space=pltpu.VMEM),
            pl.BlockSpec(memory_space=pltpu.VMEM),
        ],
        out_specs=pl.BlockSpec(memory_space=pltpu.VMEM),
        scratch_shapes=[
            pltpu.VMEM((m, n), BF16),
            pltpu.VMEM((h2, n), BF16),
            pltpu.VMEM((h4, n), BF16),
            pltpu.VMEM((h2, n), BF16),
            pltpu.VMEM((h4, n), BF16),
            pltpu.SemaphoreType.DMA((12,)),
            pltpu.SemaphoreType.DMA((12,)),
        ],
        compiler_params=pltpu.CompilerParams(
            collective_id=0,
            vmem_limit_bytes=128 * 1024 * 1024,
        ),
    )(x, w_mat)


# device time: 96828 ns/iter; 3.2369x vs baseline; 1.0461x over previous
import jax
import jax.numpy as jnp
from jax import lax
from jax.experimental import pallas as pl
from jax.experimental.pallas import tpu as pltpu

N_DEV = 4
F32 = jnp.float32
BF16 = jnp.bfloat16


def kernel(x, w_mat):
    m, k_per = x.shape
    _, n = w_mat.shape
    h2 = m // 4
    h4 = m // 8

    def body(x_ref, w_ref, out_ref, work, stage, ru1, ru2, rv1, rv2, ss, rs, os):
        out_dmas = []
        my = lax.axis_index("i")
        yp = my ^ 1
        xp = my ^ 3
        a = my // 2
        b = (my ^ (my // 2)) & 1

        barrier_sem = pltpu.get_barrier_semaphore()
        for nbr in (yp, xp):
            pl.semaphore_signal(
                barrier_sem, inc=1,
                device_id=(nbr,), device_id_type=pl.DeviceIdType.MESH,
            )
        pl.semaphore_wait(barrier_sem, 2)

        u_keep = b * h2
        u_send = (1 - b) * h2
        uq_keep = u_keep + a * h4
        uq_send = u_keep + (1 - a) * h4
        v_base = m // 2
        v_keep = v_base + a * h2
        v_send = v_base + (1 - a) * h2
        vq_keep = v_keep + b * h4
        vq_send = v_keep + (1 - b) * h4

        w_bf = w_ref[:, :].astype(BF16)

        def gemm_rows(off, nrows):
            work[pl.ds(off, nrows), :] = jnp.dot(
                x_ref[pl.ds(off, nrows), :].astype(BF16),
                w_bf,
                preferred_element_type=F32,
            ).astype(BF16)

        def copy(src_rows, nrows, dst_ref, sem_idx, partner):
            return pltpu.make_async_remote_copy(
                src_ref=work.at[pl.ds(src_rows, nrows)],
                dst_ref=dst_ref,
                send_sem=ss.at[sem_idx],
                recv_sem=rs.at[sem_idx],
                device_id=(partner,),
                device_id_type=pl.DeviceIdType.MESH,
            )

        def reduce_rows(off, nrows, recv):
            cur = work[pl.ds(off, nrows), :].astype(F32)
            work[pl.ds(off, nrows), :] = (
                cur + recv[:, :].astype(F32)
            ).astype(BF16)

        def silu_rows(off, nrows):
            y = work[pl.ds(off, nrows), :].astype(F32)
            stage[pl.ds(off, nrows), :] = y * (1.0 / (1.0 + jnp.exp(-y)))
            dma = pltpu.make_async_copy(
                stage.at[pl.ds(off, nrows)],
                out_ref.at[pl.ds(off, nrows)],
                os.at[len(out_dmas)],
            )
            dma.start()
            out_dmas.append(dma)

        gemm_rows(u_send, h2)
        s1a = copy(u_send + (1 - a) * h4, h4, ru1.at[pl.ds((1 - a) * h4, h4)], 0, yp)
        s1b = copy(u_send + a * h4, h4, ru1.at[pl.ds(a * h4, h4)], 1, yp)
        s1a.start()
        s1b.start()
        gemm_rows(v_send, h2)
        t1a = copy(v_send + (1 - b) * h4, h4, rv1.at[pl.ds((1 - b) * h4, h4)], 6, xp)
        t1b = copy(v_send + b * h4, h4, rv1.at[pl.ds(b * h4, h4)], 7, xp)
        t1a.start()
        t1b.start()
        gemm_rows(u_keep, h2)
        gemm_rows(v_keep, h2)

        s1a.wait_recv()
        reduce_rows(uq_send, h4, ru1.at[pl.ds((1 - a) * h4, h4)])
        s2 = copy(uq_send, h4, ru2, 2, xp)
        s2.start()
        t1a.wait_recv()
        reduce_rows(vq_send, h4, rv1.at[pl.ds((1 - b) * h4, h4)])
        t2 = copy(vq_send, h4, rv2, 8, yp)
        t2.start()
        s1b.wait_recv()
        reduce_rows(uq_keep, h4, ru1.at[pl.ds(a * h4, h4)])
        t1b.wait_recv()
        reduce_rows(vq_keep, h4, rv1.at[pl.ds(b * h4, h4)])

        s2.wait_recv()
        reduce_rows(uq_keep, h4, ru2)
        s3 = copy(uq_keep, h4, work.at[pl.ds(uq_keep, h4)], 3, xp)
        s4a = copy(uq_keep, h4, work.at[pl.ds(uq_keep, h4)], 4, yp)
        s3.start()
        s4a.start()
        t2.wait_recv()
        reduce_rows(vq_keep, h4, rv2)
        t3 = copy(vq_keep, h4, work.at[pl.ds(vq_keep, h4)], 9, yp)
        t4a = copy(vq_keep, h4, work.at[pl.ds(vq_keep, h4)], 10, xp)
        t3.start()
        t4a.start()
        silu_rows(uq_keep, h4)
        silu_rows(vq_keep, h4)

        r3 = copy(uq_keep, h4, work.at[pl.ds(uq_send, h4)], 3, xp)
        r3.wait_recv()
        s4b = copy(uq_send, h4, work.at[pl.ds(uq_send, h4)], 5, yp)
        s4b.start()
        silu_rows(uq_send, h4)
        q3 = copy(vq_keep, h4, work.at[pl.ds(vq_send, h4)], 9, yp)
        q3.wait_recv()
        t4b = copy(vq_send, h4, work.at[pl.ds(vq_send, h4)], 11, xp)
        t4b.start()
        silu_rows(vq_send, h4)

        r4a = copy(uq_keep, h4, work.at[pl.ds(u_send + a * h4, h4)], 4, yp)
        q4a = copy(vq_keep, h4, work.at[pl.ds(v_send + b * h4, h4)], 10, xp)
        r4b = copy(uq_keep, h4, work.at[pl.ds(u_send + (1 - a) * h4, h4)], 5, yp)
        q4b = copy(vq_keep, h4, work.at[pl.ds(v_send + (1 - b) * h4, h4)], 11, xp)
        r4a.wait_recv()
        silu_rows(u_send + a * h4, h4)
        q4a.wait_recv()
        silu_rows(v_send + b * h4, h4)
        r4b.wait_recv()
        silu_rows(u_send + (1 - a) * h4, h4)
        q4b.wait_recv()
        silu_rows(v_send + (1 - b) * h4, h4)

        for d in (s1a, s1b, t1a, t1b, s2, t2, s3, s4a, t3, t4a, s4b, t4b):
            d.wait_send()
        for d in out_dmas:
            d.wait()

    return pl.pallas_call(
        body,
        out_shape=jax.ShapeDtypeStruct((m, n), F32),
        in_specs=[
            pl.BlockSpec(memory_space=pltpu.VMEM),
            pl.BlockSpec(memory_space=pltpu.VMEM),
        ],
        out_specs=pl.BlockSpec(memory_space=pl.ANY),
        scratch_shapes=[
            pltpu.VMEM((m, n), BF16),
            pltpu.VMEM((m, n), F32),
            pltpu.VMEM((h2, n), BF16),
            pltpu.VMEM((h4, n), BF16),
            pltpu.VMEM((h2, n), BF16),
            pltpu.VMEM((h4, n), BF16),
            pltpu.SemaphoreType.DMA((12,)),
            pltpu.SemaphoreType.DMA((12,)),
            pltpu.SemaphoreType.DMA((8,)),
        ],
        compiler_params=pltpu.CompilerParams(
            collective_id=0,
            vmem_limit_bytes=128 * 1024 * 1024,
        ),
    )(x, w_mat)
